# baseline (speedup 1.0000x reference)
import jax
import jax.numpy as jnp
from jax.experimental import pallas as pl
from jax.experimental.pallas import tpu as pltpu

_B, _L, _D, _V = 1024, 200, 128, 25
_TOK_BLK = 4096
_CHUNK = 256

_TWO_OVER_PI = 0.6366197723675814
_C1H256 = 402.0
_C1H = 1.5703125
_C1L = 4.8387050628662109375e-4
_C2 = -4.371139000186241e-8
_S1, _S2, _S3 = -1.6666654611e-1, 8.3321608736e-3, -1.9515295891e-4
_K1, _K2, _K3 = 4.166664568298827e-2, -1.388731625493765e-3, 2.443315711809948e-5


def _pe_sum(start_b, end_b, denom, even):
    x = jnp.where(even, start_b, end_b) / denom
    kf = jnp.round(x * _TWO_OVER_PI)
    khf = jnp.floor(kf * (1.0 / 256.0))
    klf = kf - khf * 256.0
    d1 = x - khf * _C1H256
    d2 = d1 - klf * _C1H
    d3 = d2 - kf * _C1L
    y = d3 - kf * _C2
    z = y * y
    s = y + y * z * (_S1 + z * (_S2 + z * _S3))
    c = 1.0 + z * (-0.5 + z * (_K1 + z * (_K2 + z * _K3)))
    ki = kf.astype(jnp.int32)
    qodd = (ki & 1) != 0
    sin_x = jnp.where(qodd, c, s)
    sin_x = jnp.where((ki & 2) == 0, sin_x, -sin_x)
    cos_x = jnp.where(qodd, s, c)
    cos_x = jnp.where(((ki + 1) & 2) == 0, cos_x, -cos_x)
    return jnp.where(even,
                     sin_x + pltpu.roll(sin_x, 127, 1),
                     cos_x + pltpu.roll(cos_x, 1, 1))


def _embed_block(value_ref, chrom_ref, start_ref, end_ref, tab_ref, denom_ref,
                 out_ref):
    denom = denom_ref[...]
    tab = tab_ref[...]
    c = _CHUNK
    even = (jax.lax.broadcasted_iota(jnp.int32, (c, _D), 1) & 1) == 0
    lane32 = jax.lax.broadcasted_iota(jnp.int32, (c, 32), 1)

    def body(i, carry):
        sl = pl.ds(i * c, c)
        v = value_ref[sl, :]
        idx = chrom_ref[sl, :]
        aug = (lane32 == idx).astype(jnp.float32)
        aug = jnp.where(lane32 == _V, v, aug)
        aug = jnp.where(lane32 == _V + 1, 1.0, aug)
        base = jax.lax.dot_general(
            aug, tab, (((1,), (0,)), ((), ())),
            preferred_element_type=jnp.float32)
        pe = _pe_sum(start_ref[sl, :].astype(jnp.float32),
                     end_ref[sl, :].astype(jnp.float32), denom, even)
        out_ref[sl, :] = base + pe
        return carry

    jax.lax.fori_loop(0, _TOK_BLK // c, body, 0)


def kernel(value, chromosome, hg38_start, hg38_end, W, b, chrom_table):
    n = _B * _L
    v2 = value.reshape(n, 1)
    c2 = chromosome.reshape(n, 1).astype(jnp.int32)
    s2 = hg38_start.reshape(n, 1).astype(jnp.int32)
    e2 = hg38_end.reshape(n, 1).astype(jnp.int32)
    tab = jnp.concatenate(
        [chrom_table, W.reshape(1, _D), b.reshape(1, _D),
         jnp.zeros((32 - _V - 2, _D), jnp.float32)], axis=0)
    _2i = jnp.arange(0, _D, 2, dtype=jnp.float32)
    denom = 10000.0 ** (_2i / _D)
    denom_full = jnp.repeat(denom, 2).reshape(1, _D)

    g = n // _TOK_BLK
    tok = pl.BlockSpec((_TOK_BLK, 1), lambda i: (i, 0))
    out = pl.pallas_call(
        _embed_block,
        grid=(g,),
        in_specs=[tok, tok, tok, tok,
                  pl.BlockSpec((32, _D), lambda i: (0, 0)),
                  pl.BlockSpec((1, _D), lambda i: (0, 0))],
        out_specs=pl.BlockSpec((_TOK_BLK, _D), lambda i: (i, 0)),
        out_shape=jax.ShapeDtypeStruct((n, _D), jnp.float32),
    )(v2, c2, s2, e2, tab, denom_full)
    return out.reshape(_B, _L, _D)


# speedup vs baseline: 1.2028x; 1.2028x over previous
import jax
import jax.numpy as jnp
from jax.experimental import pallas as pl
from jax.experimental.pallas import tpu as pltpu

_B, _L, _D, _V = 1024, 200, 128, 25
_TOK_BLK = 4096
_CHUNK = 256

_TWO_OVER_PI = 0.6366197723675814
_C1H256 = 402.0
_C1H = 1.5703125
_C1L = 4.8387050628662109375e-4
_C2 = -4.371139000186241e-8
_S1, _S2, _S3 = -1.6666654611e-1, 8.3321608736e-3, -1.9515295891e-4
_K1, _K2, _K3 = 4.166664568298827e-2, -1.388731625493765e-3, 2.443315711809948e-5


def _pe_sum(start_b, end_b, denom, even):
    x = jnp.where(even, start_b, end_b) / denom
    kf = jnp.round(x * _TWO_OVER_PI)
    khf = jnp.floor(kf * (1.0 / 256.0))
    klf = kf - khf * 256.0
    d1 = x - khf * _C1H256
    d2 = d1 - klf * _C1H
    d3 = d2 - kf * _C1L
    y = d3 - kf * _C2
    z = y * y
    s = y + y * z * (_S1 + z * (_S2 + z * _S3))
    c = 1.0 + z * (-0.5 + z * (_K1 + z * (_K2 + z * _K3)))
    ki = kf.astype(jnp.int32)
    qodd = (ki & 1) != 0
    sin_x = jnp.where(qodd, c, s)
    sin_x = jnp.where((ki & 2) == 0, sin_x, -sin_x)
    cos_x = jnp.where(qodd, s, c)
    cos_x = jnp.where(((ki + 1) & 2) == 0, cos_x, -cos_x)
    return jnp.where(even,
                     sin_x + pltpu.roll(sin_x, 127, 1),
                     cos_x + pltpu.roll(cos_x, 1, 1))


def _embed_block(value_ref, chrom_ref, start_ref, end_ref, tab_ref, denom_ref,
                 out_ref):
    denom = denom_ref[...]
    tab = tab_ref[...]
    t = _TOK_BLK
    even = (jax.lax.broadcasted_iota(jnp.int32, (t, _D), 1) & 1) == 0
    out_ref[...] = _pe_sum(start_ref[...].astype(jnp.float32),
                           end_ref[...].astype(jnp.float32), denom, even)

    lane32 = jax.lax.broadcasted_iota(jnp.int32, (t, 32), 1)
    aug = (lane32 == chrom_ref[...]).astype(jnp.float32)
    aug = jnp.where(lane32 == _V, value_ref[...], aug)
    aug = jnp.where(lane32 == _V + 1, 1.0, aug)
    out_ref[...] += jax.lax.dot_general(
        aug, tab, (((1,), (0,)), ((), ())),
        preferred_element_type=jnp.float32)


def kernel(value, chromosome, hg38_start, hg38_end, W, b, chrom_table):
    n = _B * _L
    v2 = value.reshape(n, 1)
    c2 = chromosome.reshape(n, 1).astype(jnp.int32)
    s2 = hg38_start.reshape(n, 1).astype(jnp.int32)
    e2 = hg38_end.reshape(n, 1).astype(jnp.int32)
    tab = jnp.concatenate(
        [chrom_table, W.reshape(1, _D), b.reshape(1, _D),
         jnp.zeros((32 - _V - 2, _D), jnp.float32)], axis=0)
    _2i = jnp.arange(0, _D, 2, dtype=jnp.float32)
    denom = 10000.0 ** (_2i / _D)
    denom_full = jnp.repeat(denom, 2).reshape(1, _D)

    g = n // _TOK_BLK
    tok = pl.BlockSpec((_TOK_BLK, 1), lambda i: (i, 0))
    out = pl.pallas_call(
        _embed_block,
        grid=(g,),
        in_specs=[tok, tok, tok, tok,
                  pl.BlockSpec((32, _D), lambda i: (0, 0)),
                  pl.BlockSpec((1, _D), lambda i: (0, 0))],
        out_specs=pl.BlockSpec((_TOK_BLK, _D), lambda i: (i, 0)),
        out_shape=jax.ShapeDtypeStruct((n, _D), jnp.float32),
    )(v2, c2, s2, e2, tab, denom_full)
    return out.reshape(_B, _L, _D)


# speedup vs baseline: 1.3031x; 1.0833x over previous
import jax
import jax.numpy as jnp
from jax.experimental import pallas as pl
from jax.experimental.pallas import tpu as pltpu

_B, _L, _D, _V = 1024, 200, 128, 25
_TOK_BLK = 4096

_TWO_OVER_PI = 0.6366197723675814
_K256 = 0.002486796
_CMID = 0.12385966
_C1F = 1.5707964
_SA, _SB = 0.998853778, -0.159791158
_CC0, _CC1, _CC2 = 0.999987147, -0.499654263, 0.0402869076


def _pe_sum(start_b, end_b, denom, even):
    x = jnp.where(even, start_b, end_b) / denom
    khf = jnp.round(x * _K256)
    d1 = x - khf * 402.0
    xr = d1 - khf * _CMID
    klf = jnp.round(xr * _TWO_OVER_PI)
    y = xr - klf * _C1F
    z = y * y
    s = y * (_SA + _SB * z)
    c = _CC0 + z * (_CC1 + _CC2 * z)
    ki = klf.astype(jnp.int32) + 128
    qodd = (ki & 1) != 0
    sin_x = jnp.where(qodd, c, s)
    sin_x = jnp.where((ki & 2) == 0, sin_x, -sin_x)
    cos_x = jnp.where(qodd, s, c)
    cos_x = jnp.where(((ki + 1) & 2) == 0, cos_x, -cos_x)
    return jnp.where(even,
                     sin_x + pltpu.roll(sin_x, 127, 1),
                     cos_x + pltpu.roll(cos_x, 1, 1))


def _embed_block(value_ref, chrom_ref, start_ref, end_ref, tab_ref, denom_ref,
                 out_ref):
    denom = denom_ref[...]
    tab = tab_ref[...]
    t = _TOK_BLK
    even = (jax.lax.broadcasted_iota(jnp.int32, (t, _D), 1) & 1) == 0
    out_ref[...] = _pe_sum(start_ref[...].astype(jnp.float32),
                           end_ref[...].astype(jnp.float32), denom, even)

    lane32 = jax.lax.broadcasted_iota(jnp.int32, (t, 32), 1)
    aug = (lane32 == chrom_ref[...]).astype(jnp.float32)
    aug = jnp.where(lane32 == _V, value_ref[...], aug)
    aug = jnp.where(lane32 == _V + 1, 1.0, aug)
    out_ref[...] += jax.lax.dot_general(
        aug, tab, (((1,), (0,)), ((), ())),
        preferred_element_type=jnp.float32)


def kernel(value, chromosome, hg38_start, hg38_end, W, b, chrom_table):
    n = _B * _L
    v2 = value.reshape(n, 1)
    c2 = chromosome.reshape(n, 1).astype(jnp.int32)
    s2 = hg38_start.reshape(n, 1).astype(jnp.int32)
    e2 = hg38_end.reshape(n, 1).astype(jnp.int32)
    tab = jnp.concatenate(
        [chrom_table, W.reshape(1, _D), b.reshape(1, _D),
         jnp.zeros((32 - _V - 2, _D), jnp.float32)], axis=0)
    _2i = jnp.arange(0, _D, 2, dtype=jnp.float32)
    denom = 10000.0 ** (_2i / _D)
    denom_full = jnp.repeat(denom, 2).reshape(1, _D)

    g = n // _TOK_BLK
    tok = pl.BlockSpec((_TOK_BLK, 1), lambda i: (i, 0))
    out = pl.pallas_call(
        _embed_block,
        grid=(g,),
        in_specs=[tok, tok, tok, tok,
                  pl.BlockSpec((32, _D), lambda i: (0, 0)),
                  pl.BlockSpec((1, _D), lambda i: (0, 0))],
        out_specs=pl.BlockSpec((_TOK_BLK, _D), lambda i: (i, 0)),
        out_shape=jax.ShapeDtypeStruct((n, _D), jnp.float32),
    )(v2, c2, s2, e2, tab, denom_full)
    return out.reshape(_B, _L, _D)


# speedup vs baseline: 1.3191x; 1.0123x over previous
import jax
import jax.numpy as jnp
from jax.experimental import pallas as pl
from jax.experimental.pallas import tpu as pltpu

_B, _L, _D, _V = 1024, 200, 128, 25
_TOK_BLK = 4096

_TWO_OVER_PI = 0.6366197723675814
_K256 = 0.002486796
_CMID = 0.12385966
_C1F = 1.5707964
_SA, _SB = 0.998853778, -0.159791158
_CC0, _CC1, _CC2 = 0.999987147, -0.499654263, 0.0402869076


def _pe_sum(start_b, end_b, denom, even):
    x = jnp.where(even, start_b, end_b) / denom
    khf = jnp.round(x * _K256)
    d1 = x - khf * 402.0
    xr = d1 - khf * _CMID
    klf = jnp.round(xr * _TWO_OVER_PI)
    y = xr - klf * _C1F
    z = y * y
    s = y * (_SA + _SB * z)
    c = _CC0 + z * (_CC1 + _CC2 * z)
    ki = klf.astype(jnp.int32) + 128
    qodd = (ki & 1) != 0
    sin_x = jnp.where(qodd, c, s)
    sin_x = jnp.where((ki & 2) == 0, sin_x, -sin_x)
    cos_x = jnp.where(qodd, s, c)
    cos_x = jnp.where(((ki + 1) & 2) == 0, cos_x, -cos_x)
    return jnp.where(even,
                     sin_x + pltpu.roll(sin_x, 127, 1),
                     cos_x + pltpu.roll(cos_x, 1, 1))


def _embed_block(value_ref, chrom_ref, start_ref, end_ref, tab_ref, denom_ref,
                 out_ref):
    denom = denom_ref[...]
    tab = tab_ref[...]
    t = _TOK_BLK
    even = (jax.lax.broadcasted_iota(jnp.int32, (t, _D), 1) & 1) == 0
    pe = _pe_sum(start_ref[...].astype(jnp.float32),
                 end_ref[...].astype(jnp.float32), denom, even)

    lane32 = jax.lax.broadcasted_iota(jnp.int32, (t, 32), 1)
    aug = (lane32 == chrom_ref[...]).astype(jnp.float32)
    aug = jnp.where(lane32 == _V, value_ref[...], aug)
    aug = jnp.where(lane32 == _V + 1, 1.0, aug)
    out_ref[...] = pe + jax.lax.dot_general(
        aug, tab, (((1,), (0,)), ((), ())),
        preferred_element_type=jnp.float32)


def kernel(value, chromosome, hg38_start, hg38_end, W, b, chrom_table):
    n = _B * _L
    v2 = value.reshape(n, 1)
    c2 = chromosome.reshape(n, 1).astype(jnp.int32)
    s2 = hg38_start.reshape(n, 1).astype(jnp.int32)
    e2 = hg38_end.reshape(n, 1).astype(jnp.int32)
    tab = jnp.concatenate(
        [chrom_table, W.reshape(1, _D), b.reshape(1, _D),
         jnp.zeros((32 - _V - 2, _D), jnp.float32)], axis=0)
    _2i = jnp.arange(0, _D, 2, dtype=jnp.float32)
    denom = 10000.0 ** (_2i / _D)
    denom_full = jnp.repeat(denom, 2).reshape(1, _D)

    g = n // _TOK_BLK
    tok = pl.BlockSpec((_TOK_BLK, 1), lambda i: (i, 0))
    out = pl.pallas_call(
        _embed_block,
        grid=(g,),
        in_specs=[tok, tok, tok, tok,
                  pl.BlockSpec((32, _D), lambda i: (0, 0)),
                  pl.BlockSpec((1, _D), lambda i: (0, 0))],
        out_specs=pl.BlockSpec((_TOK_BLK, _D), lambda i: (i, 0)),
        out_shape=jax.ShapeDtypeStruct((n, _D), jnp.float32),
        compiler_params=pltpu.CompilerParams(
            dimension_semantics=("parallel",)),
    )(v2, c2, s2, e2, tab, denom_full)
    return out.reshape(_B, _L, _D)


# speedup vs baseline: 1.3875x; 1.0519x over previous
import jax
import jax.numpy as jnp
from jax.experimental import pallas as pl
from jax.experimental.pallas import tpu as pltpu

_B, _L, _D, _V = 1024, 200, 128, 25
_TOK_BLK = 4096

_TWO_OVER_PI = 0.6366197723675814
_K256 = 0.002486796
_CMID = 0.12385966
_C1F = 1.5707964
_SA, _SB = 0.998853778, -0.159791158
_CC0, _CC1, _CC2 = 0.999987147, -0.499654263, 0.0402869076


def _pe_sum(start_b, end_b, denom, even):
    x = jnp.where(even, start_b, end_b) / denom
    khf = jnp.round(x * _K256)
    d1 = x - khf * 402.0
    xr = d1 - khf * _CMID
    klf = jnp.round(xr * _TWO_OVER_PI)
    y = xr - klf * _C1F
    yb = y.astype(jnp.bfloat16)
    zb = yb * yb
    s = yb * (_SA + _SB * zb)
    c = _CC0 + zb * (_CC1 + _CC2 * zb)
    ki = klf.astype(jnp.int32) + 128
    qodd = (ki & 1) != 0
    sin_x = jnp.where(qodd, c, s)
    sin_x = jnp.where((ki & 2) == 0, sin_x, -sin_x)
    cos_x = jnp.where(qodd, s, c)
    cos_x = jnp.where(((ki + 1) & 2) == 0, cos_x, -cos_x)
    pe = jnp.where(even,
                   sin_x + pltpu.roll(sin_x, 127, 1),
                   cos_x + pltpu.roll(cos_x, 1, 1))
    return pe.astype(jnp.float32)


def _embed_block(value_ref, chrom_ref, start_ref, end_ref, tab_ref, denom_ref,
                 out_ref):
    denom = denom_ref[...]
    tab = tab_ref[...]
    t = _TOK_BLK
    even = (jax.lax.broadcasted_iota(jnp.int32, (t, _D), 1) & 1) == 0
    pe = _pe_sum(start_ref[...].astype(jnp.float32),
                 end_ref[...].astype(jnp.float32), denom, even)

    lane32 = jax.lax.broadcasted_iota(jnp.int32, (t, 32), 1)
    aug = (lane32 == chrom_ref[...]).astype(jnp.float32)
    aug = jnp.where(lane32 == _V, value_ref[...], aug)
    aug = jnp.where(lane32 == _V + 1, 1.0, aug)
    out_ref[...] = pe + jax.lax.dot_general(
        aug, tab, (((1,), (0,)), ((), ())),
        preferred_element_type=jnp.float32)


def kernel(value, chromosome, hg38_start, hg38_end, W, b, chrom_table):
    n = _B * _L
    v2 = value.reshape(n, 1)
    c2 = chromosome.reshape(n, 1).astype(jnp.int32)
    s2 = hg38_start.reshape(n, 1).astype(jnp.int32)
    e2 = hg38_end.reshape(n, 1).astype(jnp.int32)
    tab = jnp.concatenate(
        [chrom_table, W.reshape(1, _D), b.reshape(1, _D),
         jnp.zeros((32 - _V - 2, _D), jnp.float32)], axis=0)
    _2i = jnp.arange(0, _D, 2, dtype=jnp.float32)
    denom = 10000.0 ** (_2i / _D)
    denom_full = jnp.repeat(denom, 2).reshape(1, _D)

    g = n // _TOK_BLK
    tok = pl.BlockSpec((_TOK_BLK, 1), lambda i: (i, 0))
    out = pl.pallas_call(
        _embed_block,
        grid=(g,),
        in_specs=[tok, tok, tok, tok,
                  pl.BlockSpec((32, _D), lambda i: (0, 0)),
                  pl.BlockSpec((1, _D), lambda i: (0, 0))],
        out_specs=pl.BlockSpec((_TOK_BLK, _D), lambda i: (i, 0)),
        out_shape=jax.ShapeDtypeStruct((n, _D), jnp.float32),
        compiler_params=pltpu.CompilerParams(
            dimension_semantics=("parallel",)),
    )(v2, c2, s2, e2, tab, denom_full)
    return out.reshape(_B, _L, _D)


# speedup vs baseline: 1.4213x; 1.0243x over previous
import jax
import jax.numpy as jnp
from jax.experimental import pallas as pl
from jax.experimental.pallas import tpu as pltpu

_B, _L, _D, _V = 1024, 200, 128, 25
_TOK_BLK = 8192

_TWO_OVER_PI = 0.6366197723675814
_K256 = 0.002486796
_CMID = 0.12385966
_C1F = 1.5707964
_SA, _SB = 0.998853778, -0.159791158
_CC0, _CC1, _CC2 = 0.999987147, -0.499654263, 0.0402869076


def _pe_sum(start_b, end_b, denom, even):
    x = jnp.where(even, start_b, end_b) / denom
    khf = jnp.round(x * _K256)
    d1 = x - khf * 402.0
    xr = d1 - khf * _CMID
    klf = jnp.round(xr * _TWO_OVER_PI)
    y = xr - klf * _C1F
    yb = y.astype(jnp.bfloat16)
    zb = yb * yb
    s = yb * (_SA + _SB * zb)
    c = _CC0 + zb * (_CC1 + _CC2 * zb)
    ki = klf.astype(jnp.int32) + 128
    qodd = (ki & 1) != 0
    sin_x = jnp.where(qodd, c, s)
    sin_x = jnp.where((ki & 2) == 0, sin_x, -sin_x)
    cos_x = jnp.where(qodd, s, c)
    cos_x = jnp.where(((ki + 1) & 2) == 0, cos_x, -cos_x)
    pe = jnp.where(even,
                   sin_x + pltpu.roll(sin_x, 127, 1),
                   cos_x + pltpu.roll(cos_x, 1, 1))
    return pe.astype(jnp.float32)


def _embed_block(value_ref, chrom_ref, start_ref, end_ref, tab_ref, denom_ref,
                 out_ref):
    denom = denom_ref[...]
    tab = tab_ref[...]
    t = _TOK_BLK
    even = (jax.lax.broadcasted_iota(jnp.int32, (t, _D), 1) & 1) == 0
    pe = _pe_sum(start_ref[...].astype(jnp.float32),
                 end_ref[...].astype(jnp.float32), denom, even)

    lane32 = jax.lax.broadcasted_iota(jnp.int32, (t, 32), 1)
    aug = (lane32 == chrom_ref[...]).astype(jnp.float32)
    aug = jnp.where(lane32 == _V, value_ref[...], aug)
    aug = jnp.where(lane32 == _V + 1, 1.0, aug)
    out_ref[...] = pe + jax.lax.dot_general(
        aug, tab, (((1,), (0,)), ((), ())),
        preferred_element_type=jnp.float32)


def kernel(value, chromosome, hg38_start, hg38_end, W, b, chrom_table):
    n = _B * _L
    v2 = value.reshape(n, 1)
    c2 = chromosome.reshape(n, 1).astype(jnp.int32)
    s2 = hg38_start.reshape(n, 1).astype(jnp.int32)
    e2 = hg38_end.reshape(n, 1).astype(jnp.int32)
    tab = jnp.concatenate(
        [chrom_table, W.reshape(1, _D), b.reshape(1, _D),
         jnp.zeros((32 - _V - 2, _D), jnp.float32)], axis=0)
    _2i = jnp.arange(0, _D, 2, dtype=jnp.float32)
    denom = 10000.0 ** (_2i / _D)
    denom_full = jnp.repeat(denom, 2).reshape(1, _D)

    g = n // _TOK_BLK
    tok = pl.BlockSpec((_TOK_BLK, 1), lambda i: (i, 0))
    out = pl.pallas_call(
        _embed_block,
        grid=(g,),
        in_specs=[tok, tok, tok, tok,
                  pl.BlockSpec((32, _D), lambda i: (0, 0)),
                  pl.BlockSpec((1, _D), lambda i: (0, 0))],
        out_specs=pl.BlockSpec((_TOK_BLK, _D), lambda i: (i, 0)),
        out_shape=jax.ShapeDtypeStruct((n, _D), jnp.float32),
        compiler_params=pltpu.CompilerParams(
            dimension_semantics=("parallel",)),
    )(v2, c2, s2, e2, tab, denom_full)
    return out.reshape(_B, _L, _D)
